# Initial kernel scaffold; baseline (speedup 1.0000x reference)
import functools

import jax
import jax.numpy as jnp
from jax import lax
from jax.experimental import pallas as pl
from jax.experimental.pallas import tpu as pltpu
from jax.experimental.pallas import tpu_sc as plsc

B, T, F, D = 4096, 50, 4, 32
NC, NS = 2, 16
NW = NC * NS
PB = B // NW
NB = 8
NCHUNK = PB // NB
GW = 80
FH_PER = NB * T * F
TK_PER = NB * T
FH_K = FH_PER // GW
TK_K = TK_PER // GW

_mesh = plsc.VectorSubcoreMesh(core_axis_name="c", subcore_axis_name="s")


@functools.partial(
    pl.kernel,
    out_type=[
        jax.ShapeDtypeStruct((B, 3 * D), jnp.float32),
        jax.ShapeDtypeStruct((B * T, 2 * D), jnp.float32),
        jax.ShapeDtypeStruct((B, 2 * D), jnp.float32),
    ],
    mesh=_mesh,
    scratch_types=[
        pltpu.VMEM((FH_K, GW), jnp.int32),
        pltpu.VMEM((TK_K, GW), jnp.int32),
        pltpu.VMEM((NB,), jnp.int32),
        pltpu.VMEM((NB,), jnp.int32),
        pltpu.VMEM((NB * F,), jnp.int32),
        pltpu.VMEM((FH_PER, D), jnp.float32),
        pltpu.VMEM((TK_PER, D), jnp.float32),
        pltpu.VMEM((NB * T, 2 * D), jnp.float32),
        pltpu.VMEM((NB, 2 * D), jnp.float32),
        pltpu.VMEM((NB, D), jnp.float32),
        pltpu.VMEM((NB, D), jnp.float32),
        pltpu.VMEM((NB * F, D), jnp.float32),
        pltpu.VMEM((NB, 3 * D), jnp.float32),
        pltpu.SemaphoreType.DMA,
    ],
)
def _sc_body(fh_idx, tk_idx, nm_idx, nmh_idx, fb_idx, ftab, ttab, ntab,
             ieb_out, his_out, hsum_out,
             fidx_v, tidx_v, nidx_v, nhidx_v, fbidx_v,
             fg, tg, hisb, hsumb, n1, n2, fbg, iebb, sem):
  wid = lax.axis_index("s") * NC + lax.axis_index("c")
  wbase = wid * PB

  def subchunk(si, carry):
    base = wbase + si * NB
    k8 = base // NB
    pltpu.sync_copy(fh_idx.at[pl.ds(k8 * FH_K, FH_K)], fidx_v)
    pltpu.sync_copy(tk_idx.at[pl.ds(k8 * TK_K, TK_K)], tidx_v)
    pltpu.sync_copy(nm_idx.at[k8], nidx_v)
    pltpu.sync_copy(nmh_idx.at[k8], nhidx_v)
    pltpu.sync_copy(fb_idx.at[k8], fbidx_v)
    cps = []
    for k in range(FH_K):
      cps.append(pltpu.async_copy(ftab.at[fidx_v.at[k]],
                                  fg.at[pl.ds(k * GW, GW)], sem))
    for k in range(TK_K):
      cps.append(pltpu.async_copy(ttab.at[tidx_v.at[k]],
                                  tg.at[pl.ds(k * GW, GW)], sem))
    cps.append(pltpu.async_copy(ntab.at[nidx_v], n1, sem))
    cps.append(pltpu.async_copy(ntab.at[nhidx_v], n2, sem))
    cps.append(pltpu.async_copy(ftab.at[fbidx_v], fbg, sem))
    for cp in cps:
      cp.wait()

    zero = jnp.zeros((16,), jnp.float32)
    for b in range(NB):
      for h in range(0, 2 * D, 16):
        hsumb[b, pl.ds(h, 16)] = zero

    def pbody(p, c):
      bb = p // T
      for h in (0, 16):
        v = (fg[4 * p, pl.ds(h, 16)] + fg[4 * p + 1, pl.ds(h, 16)] +
             fg[4 * p + 2, pl.ds(h, 16)] + fg[4 * p + 3, pl.ds(h, 16)])
        hisb[p, pl.ds(h, 16)] = v
        plsc.addupdate(hsumb.at[bb, pl.ds(h, 16)], v)
        w = tg[p, pl.ds(h, 16)]
        hisb[p, pl.ds(D + h, 16)] = w
        plsc.addupdate(hsumb.at[bb, pl.ds(D + h, 16)], w)
      return c
    lax.fori_loop(0, NB * T, pbody, 0)

    for b in range(NB):
      for h in (0, 16):
        iebb[b, pl.ds(h, 16)] = n1[b, pl.ds(h, 16)]
        iebb[b, pl.ds(D + h, 16)] = n2[b, pl.ds(h, 16)]
        fv = (fbg[4 * b, pl.ds(h, 16)] + fbg[4 * b + 1, pl.ds(h, 16)] +
              fbg[4 * b + 2, pl.ds(h, 16)] + fbg[4 * b + 3, pl.ds(h, 16)])
        iebb[b, pl.ds(2 * D + h, 16)] = fv

    pltpu.sync_copy(hisb, his_out.at[pl.ds(base * T, NB * T)])
    pltpu.sync_copy(hsumb, hsum_out.at[pl.ds(base, NB)])
    pltpu.sync_copy(iebb, ieb_out.at[pl.ds(base, NB)])
    return carry

  lax.fori_loop(0, NCHUNK, subchunk, 0)


@jax.jit
def kernel(nameid_batch, funcid_batch, nameid_his_batch, funcid_his_batch,
           trackid_his_batch, nameid_emb, funcid_emb, trackid_emb):
  fh_idx = funcid_his_batch.reshape(B * T * F // GW, GW)
  tk_idx = trackid_his_batch.reshape(B * T // GW, GW)
  nm_idx = nameid_batch.reshape(B // NB, NB)
  nmh_idx = nameid_his_batch.reshape(B // NB, NB)
  fb_idx = funcid_batch.reshape(B // NB, NB * F)
  ieb, his, hsum = _sc_body(fh_idx, tk_idx, nm_idx, nmh_idx, fb_idx,
                            funcid_emb, trackid_emb, nameid_emb)
  return ieb, his.reshape(B, T, 2 * D), hsum


# speedup vs baseline: 3.6211x; 3.6211x over previous
import functools

import jax
import jax.numpy as jnp
from jax import lax
from jax.experimental import pallas as pl
from jax.experimental.pallas import tpu as pltpu
from jax.experimental.pallas import tpu_sc as plsc

B, T, F, D = 4096, 50, 4, 32
NC, NS = 2, 16
NW = NC * NS
PB = B // NW
NB = 8
NCHUNK = PB // NB
GW = 80
FH_PER = NB * T * F
TK_PER = NB * T
FH_K = FH_PER // GW
TK_K = TK_PER // GW

_mesh = plsc.VectorSubcoreMesh(core_axis_name="c", subcore_axis_name="s")


@functools.partial(
    pl.kernel,
    out_type=[
        jax.ShapeDtypeStruct((B, 3 * D), jnp.float32),
        jax.ShapeDtypeStruct((B * T, 2 * D), jnp.float32),
        jax.ShapeDtypeStruct((B, 2 * D), jnp.float32),
    ],
    mesh=_mesh,
    compiler_params=pltpu.CompilerParams(use_tc_tiling_on_sc=False),
    scratch_types=[
        pltpu.VMEM((FH_PER,), jnp.int32),
        pltpu.VMEM((TK_PER,), jnp.int32),
        pltpu.VMEM((PB,), jnp.int32),
        pltpu.VMEM((PB,), jnp.int32),
        pltpu.VMEM((PB * F,), jnp.int32),
        pltpu.VMEM((FH_PER, D), jnp.float32),
        pltpu.VMEM((TK_PER, D), jnp.float32),
        pltpu.VMEM((NB * T, 2 * D), jnp.float32),
        pltpu.VMEM((NB, 2 * D), jnp.float32),
        pltpu.VMEM((NB, D), jnp.float32),
        pltpu.VMEM((NB, D), jnp.float32),
        pltpu.VMEM((NB * F, D), jnp.float32),
        pltpu.VMEM((NB, 3 * D), jnp.float32),
        pltpu.SemaphoreType.DMA,
    ],
)
def _sc_body(fh_idx, tk_idx, nm_idx, nmh_idx, fb_idx, ftab, ttab, ntab,
             ieb_out, his_out, hsum_out,
             fidx_v, tidx_v, nidx_v, nhidx_v, fbidx_v,
             fg, tg, hisb, hsumb, n1, n2, fbg, iebb, sem):
  wid = lax.axis_index("s") * NC + lax.axis_index("c")
  wbase = wid * PB

  pltpu.sync_copy(nm_idx.at[pl.ds(wbase, PB)], nidx_v)
  pltpu.sync_copy(nmh_idx.at[pl.ds(wbase, PB)], nhidx_v)
  pltpu.sync_copy(fb_idx.at[pl.ds(wbase * F, PB * F)], fbidx_v)

  def subchunk(si, carry):
    base = wbase + si * NB
    pltpu.sync_copy(fh_idx.at[pl.ds(base * T * F, FH_PER)], fidx_v)
    pltpu.sync_copy(tk_idx.at[pl.ds(base * T, TK_PER)], tidx_v)
    cps = []
    for k in range(FH_K):
      cps.append(pltpu.async_copy(ftab.at[fidx_v.at[pl.ds(k * GW, GW)]],
                                  fg.at[pl.ds(k * GW, GW)], sem))
    for k in range(TK_K):
      cps.append(pltpu.async_copy(ttab.at[tidx_v.at[pl.ds(k * GW, GW)]],
                                  tg.at[pl.ds(k * GW, GW)], sem))
    cps.append(pltpu.async_copy(ntab.at[nidx_v.at[pl.ds(si * NB, NB)]], n1, sem))
    cps.append(pltpu.async_copy(ntab.at[nhidx_v.at[pl.ds(si * NB, NB)]], n2, sem))
    cps.append(pltpu.async_copy(ftab.at[fbidx_v.at[pl.ds(si * NB * F, NB * F)]],
                                fbg, sem))
    for cp in cps:
      cp.wait()

    zero = jnp.zeros((16,), jnp.float32)
    for b in range(NB):
      for h in range(0, 2 * D, 16):
        hsumb[b, pl.ds(h, 16)] = zero

    def pbody(p, c):
      bb = p // T
      for h in (0, 16):
        v = (fg[4 * p, pl.ds(h, 16)] + fg[4 * p + 1, pl.ds(h, 16)] +
             fg[4 * p + 2, pl.ds(h, 16)] + fg[4 * p + 3, pl.ds(h, 16)])
        hisb[p, pl.ds(h, 16)] = v
        plsc.addupdate(hsumb.at[bb, pl.ds(h, 16)], v)
        w = tg[p, pl.ds(h, 16)]
        hisb[p, pl.ds(D + h, 16)] = w
        plsc.addupdate(hsumb.at[bb, pl.ds(D + h, 16)], w)
      return c
    lax.fori_loop(0, NB * T, pbody, 0)

    for b in range(NB):
      for h in (0, 16):
        iebb[b, pl.ds(h, 16)] = n1[b, pl.ds(h, 16)]
        iebb[b, pl.ds(D + h, 16)] = n2[b, pl.ds(h, 16)]
        fv = (fbg[4 * b, pl.ds(h, 16)] + fbg[4 * b + 1, pl.ds(h, 16)] +
              fbg[4 * b + 2, pl.ds(h, 16)] + fbg[4 * b + 3, pl.ds(h, 16)])
        iebb[b, pl.ds(2 * D + h, 16)] = fv

    pltpu.sync_copy(hisb, his_out.at[pl.ds(base * T, NB * T)])
    pltpu.sync_copy(hsumb, hsum_out.at[pl.ds(base, NB)])
    pltpu.sync_copy(iebb, ieb_out.at[pl.ds(base, NB)])
    return carry

  lax.fori_loop(0, NCHUNK, subchunk, 0)


@jax.jit
def kernel(nameid_batch, funcid_batch, nameid_his_batch, funcid_his_batch,
           trackid_his_batch, nameid_emb, funcid_emb, trackid_emb):
  ieb, his, hsum = _sc_body(
      funcid_his_batch.reshape(B * T * F),
      trackid_his_batch.reshape(B * T),
      nameid_batch,
      nameid_his_batch,
      funcid_batch.reshape(B * F),
      funcid_emb, trackid_emb, nameid_emb)
  return ieb, his.reshape(B, T, 2 * D), hsum
